# baseline (speedup 1.0000x reference)
import jax
import jax.numpy as jnp
from jax import lax
from jax.experimental import pallas as pl
from jax.experimental.pallas import tpu as pltpu
from jax.experimental.pallas import tpu_sc as plsc

_NC = 2
_NS = 16
_NW = _NC * _NS
_B = 16384
_D = 32
_BPW = _B // _NW
_CHUNK = 128
_NCHUNK = _BPW // _CHUNK


def _sc_body(ni_hbm, nj_hbm, table_hbm, beta_hbm, out_hbm,
             idx_i, idx_j, rows_i, rows_j, sums_v, out_v, beta_v, sem):
    cid = lax.axis_index("c")
    sid = lax.axis_index("s")
    wid = sid * _NC + cid
    base = wid * _BPW

    pltpu.sync_copy(ni_hbm.at[wid], idx_i)
    pltpu.sync_copy(nj_hbm.at[wid], idx_j)
    pltpu.sync_copy(beta_hbm, beta_v)

    copies = []
    for j in range(_NCHUNK):
        copies.append(pltpu.async_copy(
            table_hbm.at[idx_i.at[j]],
            rows_i.at[pl.ds(j * _CHUNK, _CHUNK)], sem))
        copies.append(pltpu.async_copy(
            table_hbm.at[idx_j.at[j]],
            rows_j.at[pl.ds(j * _CHUNK, _CHUNK)], sem))
    for cp in copies:
        cp.wait()

    def row_body(r, carry):
        zi0 = rows_i[r, pl.ds(0, 16)]
        zi1 = rows_i[r, pl.ds(16, 16)]
        zj0 = rows_j[r, pl.ds(0, 16)]
        zj1 = rows_j[r, pl.ds(16, 16)]
        d0 = zi0 - zj0
        d1 = zi1 - zj1
        s2 = d0 * d0 + d1 * d1
        sums_v[r] = plsc.cumsum(s2)
        return carry

    lax.fori_loop(0, _BPW, row_body, 0)

    beta_vec = beta_v[...]
    lane = jax.lax.broadcasted_iota(jnp.int32, (16,), 0)
    last = jnp.full((16,), 15, jnp.int32)
    for g in range(_BPW // 16):
        rows = g * 16 + lane
        x = jnp.maximum(plsc.load_gather(sums_v, [rows, last]), 1e-12)
        i = plsc.bitcast(x, jnp.int32)
        i = 0x5F3759DF - lax.shift_right_arithmetic(i, 1)
        r = plsc.bitcast(i, jnp.float32)
        half = 0.5 * x
        for _ in range(3):
            r = r * (1.5 - half * r * r)
        out_v[pl.ds(g * 16, 16)] = beta_vec - x * r

    pltpu.sync_copy(out_v, out_hbm.at[pl.ds(base, _BPW)])


def kernel(node_i, node_j, table, beta):
    mesh = plsc.VectorSubcoreMesh(core_axis_name="c", subcore_axis_name="s")
    k = pl.kernel(
        _sc_body,
        out_type=jax.ShapeDtypeStruct((_B,), jnp.float32),
        mesh=mesh,
        compiler_params=pltpu.CompilerParams(
            needs_layout_passes=False, use_tc_tiling_on_sc=False),
        scratch_types=[
            pltpu.VMEM((_NCHUNK, _CHUNK), jnp.int32),
            pltpu.VMEM((_NCHUNK, _CHUNK), jnp.int32),
            pltpu.VMEM((_BPW, _D), jnp.float32),
            pltpu.VMEM((_BPW, _D), jnp.float32),
            pltpu.VMEM((_BPW, 16), jnp.float32),
            pltpu.VMEM((_BPW,), jnp.float32),
            pltpu.VMEM((16,), jnp.float32),
            pltpu.SemaphoreType.DMA,
        ],
    )
    ni = node_i.reshape(_NW, _NCHUNK, _CHUNK)
    nj = node_j.reshape(_NW, _NCHUNK, _CHUNK)
    beta_vec = jnp.broadcast_to(beta.astype(jnp.float32), (16,))
    return k(ni, nj, table, beta_vec)
